# baseline (speedup 1.0000x reference)
import functools

import jax
import jax.numpy as jnp
from jax import lax
from jax.experimental import pallas as pl
from jax.experimental.pallas import tpu as pltpu, tpu_sc as plsc

NUM_EMBEDDINGS = 1000000
EMBEDDING_DIM = 64
BATCH = 4
SEQ_LEN = 8192

_INFO = plsc.get_sparse_core_info()
_NC, _NS = _INFO.num_cores, _INFO.num_subcores
_NW = _NC * _NS
_B = BATCH * SEQ_LEN
_B_PER_W = _B // _NW
_SEQ_PER_W = SEQ_LEN // (_NW // BATCH)
_W = 256
_NWIN = _B_PER_W // _W


def _make_gather():
    mesh = plsc.VectorSubcoreMesh(core_axis_name="c", subcore_axis_name="s")

    @functools.partial(
        pl.kernel,
        mesh=mesh,
        out_type=jax.ShapeDtypeStruct((_B, EMBEDDING_DIM), jnp.float32),
        scratch_types=[
            pltpu.VMEM((_B_PER_W,), jnp.int32),
            pltpu.VMEM((_W, EMBEDDING_DIM), jnp.float32),
            pltpu.VMEM((_W, EMBEDDING_DIM), jnp.float32),
            pltpu.SemaphoreType.DMA,
            pltpu.SemaphoreType.DMA,
            pltpu.SemaphoreType.DMA,
            pltpu.SemaphoreType.DMA,
        ],
    )
    def gather_kernel(
        table_hbm, ids_hbm, out_hbm, idx_v, stage0, stage1, g0, g1, s0, s1
    ):
        wid = lax.axis_index("s") * _NC + lax.axis_index("c")
        bat = wid // (_NW // BATCH)
        seq0 = (wid % (_NW // BATCH)) * _SEQ_PER_W
        obase = wid * _B_PER_W

        pltpu.sync_copy(ids_hbm.at[bat, pl.ds(seq0, _B_PER_W)], idx_v)

        stages = (stage0, stage1)
        gsem = (g0, g1)
        ssem = (s0, s1)

        def fire_rows(w, p):
            stage = stages[p]
            for g in range(_W // 16):
                base = pl.multiple_of(w * _W + g * 16, 16)
                ids16 = idx_v[pl.ds(base, 16)]
                for l in range(16):
                    t = ids16[l]
                    i = g * 16 + l
                    pltpu.async_copy(
                        table_hbm.at[pl.ds(t, 1)],
                        stage.at[pl.ds(i, 1)],
                        gsem[p],
                    )

        def drain_rows(p):
            pltpu.make_async_copy(
                table_hbm.at[pl.ds(0, _W)], stages[p], gsem[p]
            ).wait()

        def writeback_desc(w, p):
            dst = out_hbm.at[pl.ds(pl.multiple_of(obase + w * _W, _W), _W)]
            return pltpu.make_async_copy(stages[p], dst, ssem[p])

        fire_rows(0, 0)
        fire_rows(1, 1)

        def win_body(it, _):
            w = it * 2
            for p in range(2):
                drain_rows(p)
                writeback_desc(w + p, p).start()

                @pl.when(w + p + 2 < _NWIN)
                def _next():
                    writeback_desc(w + p, p).wait()
                    fire_rows(w + p + 2, p)

            return _

        lax.fori_loop(0, _NWIN // 2, win_body, None)
        writeback_desc(_NWIN - 2, 0).wait()
        writeback_desc(_NWIN - 1, 1).wait()

    return gather_kernel


_gather = _make_gather()


def kernel(token_ids, embedding_matrix):
    rows = _gather(embedding_matrix, token_ids.astype(jnp.int32))
    return rows.reshape(BATCH, SEQ_LEN, EMBEDDING_DIM)


# speedup vs baseline: 1.4393x; 1.4393x over previous
import functools

import jax
import jax.numpy as jnp
from jax import lax
from jax.experimental import pallas as pl
from jax.experimental.pallas import tpu as pltpu, tpu_sc as plsc

NUM_EMBEDDINGS = 1000000
EMBEDDING_DIM = 64
BATCH = 4
SEQ_LEN = 8192

_INFO = plsc.get_sparse_core_info()
_NC, _NS = _INFO.num_cores, _INFO.num_subcores
_NW = _NC * _NS
_B = BATCH * SEQ_LEN
_B_PER_W = _B // _NW
_SEQ_PER_W = SEQ_LEN // (_NW // BATCH)
_W = 256
_NWIN = _B_PER_W // _W


def _make_gather():
    mesh = plsc.VectorSubcoreMesh(core_axis_name="c", subcore_axis_name="s")

    @functools.partial(
        pl.kernel,
        mesh=mesh,
        out_type=jax.ShapeDtypeStruct((_B, EMBEDDING_DIM), jnp.float32),
        scratch_types=[
            pltpu.VMEM((_B_PER_W,), jnp.int32),
            pltpu.VMEM((_W, EMBEDDING_DIM), jnp.float32),
            pltpu.VMEM((_W, EMBEDDING_DIM), jnp.float32),
            pltpu.SemaphoreType.DMA,
            pltpu.SemaphoreType.DMA,
            pltpu.SemaphoreType.DMA,
            pltpu.SemaphoreType.DMA,
        ],
    )
    def gather_kernel(
        table_hbm, ids_hbm, out_hbm, idx_v, stage0, stage1, g0, g1, s0, s1
    ):
        wid = lax.axis_index("s") * _NC + lax.axis_index("c")
        bat = wid // (_NW // BATCH)
        seq0 = (wid % (_NW // BATCH)) * _SEQ_PER_W
        obase = wid * _B_PER_W

        pltpu.sync_copy(ids_hbm.at[bat, pl.ds(seq0, _B_PER_W)], idx_v)

        stages = (stage0, stage1)
        gsem = (g0, g1)
        ssem = (s0, s1)

        def fire_rows(w, p):
            stage = stages[p]
            for g in range(_W // 16):
                base = pl.multiple_of(w * _W + g * 16, 16)
                ids16 = idx_v[pl.ds(base, 16)]
                for l in range(16):
                    t = ids16[l]
                    i = g * 16 + l
                    pltpu.async_copy(
                        table_hbm.at[lax.shift_right_logical(t, 3), t & 7],
                        stage.at[i],
                        gsem[p],
                    )

        def drain_rows(p):
            pltpu.make_async_copy(
                out_hbm.at[pl.ds(0, _W)], stages[p], gsem[p]
            ).wait()

        def writeback_desc(w, p):
            dst = out_hbm.at[pl.ds(pl.multiple_of(obase + w * _W, _W), _W)]
            return pltpu.make_async_copy(stages[p], dst, ssem[p])

        fire_rows(0, 0)
        fire_rows(1, 1)

        def win_body(it, _):
            w = it * 2
            for p in range(2):
                drain_rows(p)
                writeback_desc(w + p, p).start()

                @pl.when(w + p + 2 < _NWIN)
                def _next():
                    writeback_desc(w + p, p).wait()
                    fire_rows(w + p + 2, p)

            return _

        lax.fori_loop(0, _NWIN // 2, win_body, None)
        writeback_desc(_NWIN - 2, 0).wait()
        writeback_desc(_NWIN - 1, 1).wait()

    return gather_kernel


_gather = _make_gather()


def kernel(token_ids, embedding_matrix):
    table3 = embedding_matrix.reshape(NUM_EMBEDDINGS // 8, 8, EMBEDDING_DIM)
    rows = _gather(table3, token_ids.astype(jnp.int32))
    return rows.reshape(BATCH, SEQ_LEN, EMBEDDING_DIM)
